# Initial kernel scaffold; baseline (speedup 1.0000x reference)
import jax
import jax.numpy as jnp
from jax.experimental import pallas as pl


def kernel(h, z, pos, mask, e_feat, z_emb, peW1, peb1, peW2, peb2, peW3, peb3, gW1, gb1, gW2, gb2, gW3, gb3, oW1, ob1, oW2, ob2):
    raise NotImplementedError("write your pallas kernel here")


# speedup vs baseline: 3.4741x; 3.4741x over previous
import jax
import jax.numpy as jnp
from jax.experimental import pallas as pl
from jax.experimental.pallas import tpu as pltpu

B, N, H = 8, 128, 128
NE, EDIM = 64, 32
ZDIM, MAXZ = 32, 100
RBF_DIM = 32
GH, SD, OD = 256, 128, 128
CUTOFF = 6.0
PMAX = 256
PBLK = 32
NB = PMAX // PBLK

_INF_BITS = 0x7F800000


def _silu(x):
    return x * jax.lax.logistic(x)


def _sel_kernel(pos_ref, posT_ref, mask_ref, maskT_ref, code_ref, frank_ref, fcode_ref):
    px = pos_ref[0][:, 0:1]
    py = pos_ref[0][:, 1:2]
    pz = pos_ref[0][:, 2:3]
    pxr = posT_ref[0][0:1, :]
    pyr = posT_ref[0][1:2, :]
    pzr = posT_ref[0][2:3, :]
    dx = px - pxr
    dy = py - pyr
    dz = pz - pzr
    d2 = (dx * dx + dy * dy) + dz * dz
    d = jnp.sqrt(jnp.maximum(d2, 0.0))
    r0_row = d[0:1, :]
    r0_col = d[:, 0:1]
    kIi = jax.lax.broadcasted_iota(jnp.int32, (N, N), 1)
    jIi = jax.lax.broadcasted_iota(jnp.int32, (N, N), 0)
    kI = kIi.astype(jnp.float32)
    jI = jIi.astype(jnp.float32)
    m_row = mask_ref[0][0:1, :]
    m_col = maskT_ref[0][:, 0:1]
    valid_row = (m_row > 0.0) & (r0_row <= CUTOFF) & (kIi[0:1, :] != 0)
    valid_col = (m_col > 0.0) & (r0_col <= CUTOFF) & (jIi[:, 0:1] != 0)
    pv = valid_col & valid_row & (jIi < kIi)
    score = (r0_col + r0_row) + 0.5 * d
    scoref = jnp.where(pv, score, jnp.inf)
    bits = jax.lax.bitcast_convert_type(scoref, jnp.int32)

    def bs_body(_, lohi):
        lo, hi = lohi
        mid = lo + (hi - lo) // 2
        cnt = jnp.sum((bits <= mid).astype(jnp.float32))
        ge = cnt >= float(PMAX)
        return (jnp.where(ge, lo, mid + 1), jnp.where(ge, mid, hi))

    lo0 = jnp.int32(0)
    hi0 = jnp.int32(_INF_BITS)
    _, tstar = jax.lax.fori_loop(0, 31, bs_body, (lo0, hi0))

    strict = (bits < tstar).astype(jnp.float32)
    tie = (bits == tstar).astype(jnp.float32)
    n_strict = jnp.sum(strict)

    up_strict = (jIi < kIi).astype(jnp.float32)
    lo_strict = (kIi < jIi).astype(jnp.float32)

    def excl_rank(m):
        within = jax.lax.dot(m, up_strict, precision=jax.lax.Precision.HIGHEST)
        above = jax.lax.dot(lo_strict, m, precision=jax.lax.Precision.HIGHEST)
        row_off = jnp.sum(above, axis=1, keepdims=True)
        return row_off + within

    tie_rank = excl_rank(tie)
    sel = strict + tie * (tie_rank < (float(PMAX) - n_strict)).astype(jnp.float32)
    rank = excl_rank(sel)

    pvf = pv.astype(jnp.float32)
    code = jI * 512.0 + kI * 2.0 + pvf
    rankm = jnp.where(sel > 0.0, rank, 1e9)
    codem = sel * code

    for j in range(N):
        frank_ref[0:1, j * N:(j + 1) * N] = rankm[j:j + 1, :]
        fcode_ref[0:1, j * N:(j + 1) * N] = codem[j:j + 1, :]

    s_col = jax.lax.broadcasted_iota(jnp.int32, (PMAX, 1), 0).astype(jnp.float32)
    pd = jnp.zeros((PMAX, 1), jnp.float32)
    CH = 1024
    for blk in range(N * N // CH):
        rch = frank_ref[0:1, blk * CH:(blk + 1) * CH]
        cch = fcode_ref[0:1, blk * CH:(blk + 1) * CH]
        oh = (rch == s_col).astype(jnp.float32)
        pd = pd + jnp.sum(oh * cch, axis=1, keepdims=True)
    code_ref[0] = pd


def _agg_kernel(code_ref, h_ref, pos_ref, z_ref, zemb_ref, efeat_ref,
                peW1_ref, peb1_ref, peW2_ref, peb2_ref, peW3_ref, peb3_ref,
                gW1_ref, gb1_ref, gW2_ref, gb2_ref, gW3_ref, gb3_ref,
                oW1_ref, ob1_ref, oW2_ref, ob2_ref,
                out_ref,
                prezj_s, prezk_s, pree_s, prej_s, prek_s, wg_s, agg_s):
    pblk = pl.program_id(1)
    bi = pl.program_id(0)
    P = jax.lax.Precision.HIGHEST

    @pl.when((pblk == 0) & (bi == 0))
    def _tables():
        zemb = zemb_ref[...]
        prezj_s[...] = jax.lax.dot(zemb, peW1_ref[0:ZDIM, :], precision=P)
        prezk_s[...] = jax.lax.dot(zemb, peW1_ref[ZDIM:2 * ZDIM, :], precision=P)
        pree_s[...] = jax.lax.dot(efeat_ref[...], peW1_ref[2 * ZDIM:3 * ZDIM, :],
                                  precision=P) + peb1_ref[...]

    @pl.when(pblk == 0)
    def _per_batch():
        codes = code_ref[0]
        jf = jnp.floor(codes / 512.0)
        rem = codes - jf * 512.0
        kf = jnp.floor(rem / 2.0)
        vf = rem - kf * 2.0
        lane128 = jax.lax.broadcasted_iota(jnp.int32, (PMAX, N), 1).astype(jnp.float32)
        ohj = (jf == lane128).astype(jnp.float32)
        ohk = (kf == lane128).astype(jnp.float32)
        posb = pos_ref[0]
        p0 = posb[0:1, :]
        pj = jax.lax.dot(ohj, posb, precision=P)
        pk = jax.lax.dot(ohk, posb, precision=P)
        vj = pj - p0
        vk = pk - p0
        vjk = pk - pj
        r0j = jnp.sqrt(jnp.sum(vj * vj, axis=1, keepdims=True))
        r0k = jnp.sqrt(jnp.sum(vk * vk, axis=1, keepdims=True))
        rjk = jnp.sqrt(jnp.sum(vjk * vjk, axis=1, keepdims=True))
        dotjk = jnp.sum(vj * vk, axis=1, keepdims=True)
        cosang = jnp.clip(dotjk / (jnp.maximum(r0j, 1e-8) * jnp.maximum(r0k, 1e-8)),
                          -1.0, 1.0)

        delta = CUTOFF / (RBF_DIM - 1)
        gamma = 1.0 / (delta * delta + 1e-12)
        cent = jax.lax.broadcasted_iota(jnp.int32, (1, RBF_DIM), 1).astype(jnp.float32) * delta

        def rbf(r):
            rc = jnp.minimum(r, CUTOFF)
            t = rc - cent
            return jnp.exp(-gamma * (t * t))

        f0j = rbf(r0j)
        f0k = rbf(r0k)
        fjk = rbf(rjk)
        lane8 = jax.lax.broadcasted_iota(jnp.int32, (PMAX, 8), 1)
        cos8 = jnp.where(lane8 < 1, cosang, 0.0)

        bf = jnp.bfloat16
        f32 = jnp.float32
        hb = h_ref[0]
        ohj_bf = ohj.astype(bf)
        ohk_bf = ohk.astype(bf)
        hj = jax.lax.dot(ohj_bf, hb, preferred_element_type=f32)
        hk = jax.lax.dot(ohk_bf, hb, preferred_element_type=f32)
        x = (jax.lax.dot(hj.astype(bf), gW1_ref[0:128, :], preferred_element_type=f32)
             + jax.lax.dot(hk.astype(bf), gW1_ref[128:256, :], preferred_element_type=f32)
             + jax.lax.dot(f0j.astype(bf), gW1_ref[256:288, :], preferred_element_type=f32)
             + jax.lax.dot(f0k.astype(bf), gW1_ref[288:320, :], preferred_element_type=f32)
             + jax.lax.dot(fjk.astype(bf), gW1_ref[320:352, :], preferred_element_type=f32)
             + jax.lax.dot(cos8.astype(bf), gW1_ref[352:360, :], preferred_element_type=f32)
             + gb1_ref[...])
        x = _silu(x)
        x = _silu(jax.lax.dot(x.astype(bf), gW2_ref[...], preferred_element_type=f32) + gb2_ref[...])
        g = jax.lax.dot(x.astype(bf), gW3_ref[...], preferred_element_type=f32) + gb3_ref[...]

        def cut(r):
            c = 0.5 * (jnp.cos(jnp.pi * r / CUTOFF) + 1.0)
            return c * (r <= CUTOFF).astype(jnp.float32)

        env = cut(r0j) * cut(r0k) * cut(rjk) * vf
        wg_s[...] = g * env

        zb = z_ref[0]
        zj = jax.lax.dot(ohj, zb, precision=P)
        zk = jax.lax.dot(ohk, zb, precision=P)
        ohzj = (zj == lane128).astype(jnp.float32)
        ohzk = (zk == lane128).astype(jnp.float32)
        prej_s[...] = jax.lax.dot(ohzj, prezj_s[...], precision=P)
        prek_s[...] = jax.lax.dot(ohzk, prezk_s[...], precision=P)
        agg_s[...] = jnp.zeros((NE, SD), jnp.float32)

    x12 = prej_s[pl.ds(pblk * PBLK, PBLK), :] + prek_s[pl.ds(pblk * PBLK, PBLK), :]
    a3 = jnp.reshape(x12, (PBLK, 1, GH)) + jnp.reshape(pree_s[...], (1, NE, GH))
    a = _silu(jnp.reshape(a3, (PBLK * NE, GH)))
    h1 = _silu(jax.lax.dot(a.astype(jnp.bfloat16), peW2_ref[...],
                           preferred_element_type=jnp.float32) + peb2_ref[...])
    sc = jax.lax.dot(h1.astype(jnp.bfloat16), peW3_ref[...],
                     preferred_element_type=jnp.float32) + peb3_ref[...]
    sc3 = jnp.reshape(sc, (PBLK, NE, SD))
    wgb = jnp.reshape(wg_s[pl.ds(pblk * PBLK, PBLK), :], (PBLK, 1, SD))
    agg_s[...] += jnp.sum(sc3 * wgb, axis=0)

    @pl.when(pblk == NB - 1)
    def _final():
        t = _silu(jax.lax.dot(agg_s[...], oW1_ref[...], precision=P) + ob1_ref[...])
        out_ref[0] = jax.lax.dot(t, oW2_ref[...], precision=P) + ob2_ref[...]


@jax.jit
def kernel(h, z, pos, mask, e_feat, z_emb, peW1, peb1, peW2, peb2, peW3, peb3,
           gW1, gb1, gW2, gb2, gW3, gb3, oW1, ob1, oW2, ob2):
    posT = jnp.transpose(pos, (0, 2, 1))
    mask_f = mask.astype(jnp.float32).reshape(B, 1, N)
    maskT_f = mask.astype(jnp.float32).reshape(B, N, 1)

    codes = pl.pallas_call(
        _sel_kernel,
        grid=(B,),
        in_specs=[
            pl.BlockSpec((1, N, 3), lambda b: (b, 0, 0)),
            pl.BlockSpec((1, 3, N), lambda b: (b, 0, 0)),
            pl.BlockSpec((1, 1, N), lambda b: (b, 0, 0)),
            pl.BlockSpec((1, N, 1), lambda b: (b, 0, 0)),
        ],
        out_specs=pl.BlockSpec((1, PMAX, 1), lambda b: (b, 0, 0)),
        out_shape=jax.ShapeDtypeStruct((B, PMAX, 1), jnp.float32),
        scratch_shapes=[
            pltpu.VMEM((1, N * N), jnp.float32),
            pltpu.VMEM((1, N * N), jnp.float32),
        ],
        compiler_params=pltpu.CompilerParams(
            dimension_semantics=("arbitrary",)),
    )(pos, posT, mask_f, maskT_f)

    posp = jnp.pad(pos, ((0, 0), (0, 0), (0, 5)))
    z_col = z.astype(jnp.float32).reshape(B, N, 1)
    zemb_p = jnp.pad(z_emb, ((0, N - (MAXZ + 1)), (0, 0)))
    gW1p = jnp.pad(gW1, ((0, 360 - gW1.shape[0]), (0, 0)))
    pe_in = 3 * ZDIM
    r2 = lambda v: v.reshape(1, -1)

    wspec = lambda shape: pl.BlockSpec(shape, lambda b, p: (0,) * len(shape))
    out = pl.pallas_call(
        _agg_kernel,
        grid=(B, NB),
        in_specs=[
            pl.BlockSpec((1, PMAX, 1), lambda b, p: (b, 0, 0)),
            pl.BlockSpec((1, N, H), lambda b, p: (b, 0, 0)),
            pl.BlockSpec((1, N, 8), lambda b, p: (b, 0, 0)),
            pl.BlockSpec((1, N, 1), lambda b, p: (b, 0, 0)),
            wspec((N, ZDIM)),
            wspec((NE, EDIM)),
            wspec((pe_in, GH)), wspec((1, GH)),
            wspec((GH, GH)), wspec((1, GH)),
            wspec((GH, SD)), wspec((1, SD)),
            wspec((360, GH)), wspec((1, GH)),
            wspec((GH, GH)), wspec((1, GH)),
            wspec((GH, SD)), wspec((1, SD)),
            wspec((SD, GH)), wspec((1, GH)),
            wspec((GH, OD)), wspec((1, OD)),
        ],
        out_specs=pl.BlockSpec((1, NE, OD), lambda b, p: (b, 0, 0)),
        out_shape=jax.ShapeDtypeStruct((B, NE, OD), jnp.float32),
        scratch_shapes=[
            pltpu.VMEM((N, GH), jnp.float32),
            pltpu.VMEM((N, GH), jnp.float32),
            pltpu.VMEM((NE, GH), jnp.float32),
            pltpu.VMEM((PMAX, GH), jnp.float32),
            pltpu.VMEM((PMAX, GH), jnp.float32),
            pltpu.VMEM((PMAX, SD), jnp.float32),
            pltpu.VMEM((NE, SD), jnp.float32),
        ],
        compiler_params=pltpu.CompilerParams(
            dimension_semantics=("arbitrary", "arbitrary")),
    )(codes, h.astype(jnp.bfloat16), posp, z_col, zemb_p, e_feat,
      peW1, r2(peb1), peW2.astype(jnp.bfloat16), r2(peb2),
      peW3.astype(jnp.bfloat16), r2(peb3),
      gW1p.astype(jnp.bfloat16), r2(gb1), gW2.astype(jnp.bfloat16), r2(gb2),
      gW3.astype(jnp.bfloat16), r2(gb3),
      oW1, r2(ob1), oW2, r2(ob2))
    return out


# speedup vs baseline: 3.5563x; 1.0237x over previous
import jax
import jax.numpy as jnp
from jax.experimental import pallas as pl
from jax.experimental.pallas import tpu as pltpu

B, N, H = 8, 128, 128
NE, EDIM = 64, 32
ZDIM, MAXZ = 32, 100
RBF_DIM = 32
GH, SD, OD = 256, 128, 128
CUTOFF = 6.0
PMAX = 256
PBLK = 32
NB = PMAX // PBLK

_INF_BITS = 0x7F800000


def _silu(x):
    return x * jax.lax.logistic(x)


def _sel_kernel(pos_ref, posT_ref, mask_ref, maskT_ref, code_ref, frank_ref, fcode_ref):
    px = pos_ref[0][:, 0:1]
    py = pos_ref[0][:, 1:2]
    pz = pos_ref[0][:, 2:3]
    pxr = posT_ref[0][0:1, :]
    pyr = posT_ref[0][1:2, :]
    pzr = posT_ref[0][2:3, :]
    dx = px - pxr
    dy = py - pyr
    dz = pz - pzr
    d2 = (dx * dx + dy * dy) + dz * dz
    d = jnp.sqrt(jnp.maximum(d2, 0.0))
    r0_row = d[0:1, :]
    r0_col = d[:, 0:1]
    kIi = jax.lax.broadcasted_iota(jnp.int32, (N, N), 1)
    jIi = jax.lax.broadcasted_iota(jnp.int32, (N, N), 0)
    kI = kIi.astype(jnp.float32)
    jI = jIi.astype(jnp.float32)
    m_row = mask_ref[0][0:1, :]
    m_col = maskT_ref[0][:, 0:1]
    valid_row = (m_row > 0.0) & (r0_row <= CUTOFF) & (kIi[0:1, :] != 0)
    valid_col = (m_col > 0.0) & (r0_col <= CUTOFF) & (jIi[:, 0:1] != 0)
    pv = valid_col & valid_row & (jIi < kIi)
    score = (r0_col + r0_row) + 0.5 * d
    scoref = jnp.where(pv, score, jnp.inf)
    bits = jax.lax.bitcast_convert_type(scoref, jnp.int32)

    def bs_body(_, lohi):
        lo, hi = lohi
        mid = lo + (hi - lo) // 2
        cnt = jnp.sum((bits <= mid).astype(jnp.float32))
        ge = cnt >= float(PMAX)
        return (jnp.where(ge, lo, mid + 1), jnp.where(ge, mid, hi))

    lo0 = jnp.int32(0)
    hi0 = jnp.int32(_INF_BITS)
    _, tstar = jax.lax.fori_loop(0, 31, bs_body, (lo0, hi0))

    strict = (bits < tstar).astype(jnp.float32)
    tie = (bits == tstar).astype(jnp.float32)
    n_strict = jnp.sum(strict)

    up_strict = (jIi < kIi).astype(jnp.float32)
    lo_strict = (kIi < jIi).astype(jnp.float32)

    def excl_rank(m):
        within = jax.lax.dot(m, up_strict, precision=jax.lax.Precision.HIGHEST)
        above = jax.lax.dot(lo_strict, m, precision=jax.lax.Precision.HIGHEST)
        row_off = jnp.sum(above, axis=1, keepdims=True)
        return row_off + within

    tie_rank = excl_rank(tie)
    sel = strict + tie * (tie_rank < (float(PMAX) - n_strict)).astype(jnp.float32)
    rank = excl_rank(sel)

    pvf = pv.astype(jnp.float32)
    code = jI * 512.0 + kI * 2.0 + pvf
    rankm = jnp.where(sel > 0.0, rank, 1e9)
    codem = sel * code

    for j in range(N):
        frank_ref[0:1, j * N:(j + 1) * N] = rankm[j:j + 1, :]
        fcode_ref[0:1, j * N:(j + 1) * N] = codem[j:j + 1, :]

    sl_col = jax.lax.broadcasted_iota(jnp.int32, (32, 1), 0).astype(jnp.float32)
    CH = 2048
    PX = jax.lax.Precision.HIGHEST
    acc = [jnp.zeros((32, 1), jnp.float32) for _ in range(8)]
    for blk in range(N * N // CH):
        rch = frank_ref[0:1, blk * CH:(blk + 1) * CH]
        cch = fcode_ref[0:1, blk * CH:(blk + 1) * CH]
        hi = jnp.floor(rch * (1.0 / 32.0))
        lo = rch - 32.0 * hi
        oh_lo = (lo == sl_col).astype(jnp.float32)
        for sh in range(8):
            msk = jnp.where(hi == float(sh), cch, 0.0)
            acc[sh] = acc[sh] + jax.lax.dot_general(
                oh_lo, msk, (((1,), (1,)), ((), ())), precision=PX)
    for sh in range(8):
        code_ref[0, sh * 32:(sh + 1) * 32, :] = acc[sh]


def _agg_kernel(code_ref, h_ref, pos_ref, z_ref, zemb_ref, efeat_ref,
                peW1_ref, peb1_ref, peW2_ref, peb2_ref, peW3_ref, peb3_ref,
                gW1_ref, gb1_ref, gW2_ref, gb2_ref, gW3_ref, gb3_ref,
                oW1_ref, ob1_ref, oW2_ref, ob2_ref,
                out_ref,
                prezj_s, prezk_s, pree_s, prej_s, prek_s, wg_s, agg_s):
    pblk = pl.program_id(1)
    bi = pl.program_id(0)
    P = jax.lax.Precision.HIGHEST

    @pl.when((pblk == 0) & (bi == 0))
    def _tables():
        zemb = zemb_ref[...]
        prezj_s[...] = jax.lax.dot(zemb, peW1_ref[0:ZDIM, :], precision=P)
        prezk_s[...] = jax.lax.dot(zemb, peW1_ref[ZDIM:2 * ZDIM, :], precision=P)
        pree_s[...] = jax.lax.dot(efeat_ref[...], peW1_ref[2 * ZDIM:3 * ZDIM, :],
                                  precision=P) + peb1_ref[...]

    @pl.when(pblk == 0)
    def _per_batch():
        codes = code_ref[0]
        jf = jnp.floor(codes / 512.0)
        rem = codes - jf * 512.0
        kf = jnp.floor(rem / 2.0)
        vf = rem - kf * 2.0
        lane128 = jax.lax.broadcasted_iota(jnp.int32, (PMAX, N), 1).astype(jnp.float32)
        ohj = (jf == lane128).astype(jnp.float32)
        ohk = (kf == lane128).astype(jnp.float32)
        posb = pos_ref[0]
        p0 = posb[0:1, :]
        pj = jax.lax.dot(ohj, posb, precision=P)
        pk = jax.lax.dot(ohk, posb, precision=P)
        vj = pj - p0
        vk = pk - p0
        vjk = pk - pj
        r0j = jnp.sqrt(jnp.sum(vj * vj, axis=1, keepdims=True))
        r0k = jnp.sqrt(jnp.sum(vk * vk, axis=1, keepdims=True))
        rjk = jnp.sqrt(jnp.sum(vjk * vjk, axis=1, keepdims=True))
        dotjk = jnp.sum(vj * vk, axis=1, keepdims=True)
        cosang = jnp.clip(dotjk / (jnp.maximum(r0j, 1e-8) * jnp.maximum(r0k, 1e-8)),
                          -1.0, 1.0)

        delta = CUTOFF / (RBF_DIM - 1)
        gamma = 1.0 / (delta * delta + 1e-12)
        cent = jax.lax.broadcasted_iota(jnp.int32, (1, RBF_DIM), 1).astype(jnp.float32) * delta

        def rbf(r):
            rc = jnp.minimum(r, CUTOFF)
            t = rc - cent
            return jnp.exp(-gamma * (t * t))

        f0j = rbf(r0j)
        f0k = rbf(r0k)
        fjk = rbf(rjk)
        lane8 = jax.lax.broadcasted_iota(jnp.int32, (PMAX, 8), 1)
        cos8 = jnp.where(lane8 < 1, cosang, 0.0)

        bf = jnp.bfloat16
        f32 = jnp.float32
        hb = h_ref[0]
        ohj_bf = ohj.astype(bf)
        ohk_bf = ohk.astype(bf)
        hj = jax.lax.dot(ohj_bf, hb, preferred_element_type=f32)
        hk = jax.lax.dot(ohk_bf, hb, preferred_element_type=f32)
        x = (jax.lax.dot(hj.astype(bf), gW1_ref[0:128, :], preferred_element_type=f32)
             + jax.lax.dot(hk.astype(bf), gW1_ref[128:256, :], preferred_element_type=f32)
             + jax.lax.dot(f0j.astype(bf), gW1_ref[256:288, :], preferred_element_type=f32)
             + jax.lax.dot(f0k.astype(bf), gW1_ref[288:320, :], preferred_element_type=f32)
             + jax.lax.dot(fjk.astype(bf), gW1_ref[320:352, :], preferred_element_type=f32)
             + jax.lax.dot(cos8.astype(bf), gW1_ref[352:360, :], preferred_element_type=f32)
             + gb1_ref[...])
        x = _silu(x)
        x = _silu(jax.lax.dot(x.astype(bf), gW2_ref[...], preferred_element_type=f32) + gb2_ref[...])
        g = jax.lax.dot(x.astype(bf), gW3_ref[...], preferred_element_type=f32) + gb3_ref[...]

        def cut(r):
            c = 0.5 * (jnp.cos(jnp.pi * r / CUTOFF) + 1.0)
            return c * (r <= CUTOFF).astype(jnp.float32)

        env = cut(r0j) * cut(r0k) * cut(rjk) * vf
        wg_s[...] = g * env

        zb = z_ref[0]
        zj = jax.lax.dot(ohj, zb, precision=P)
        zk = jax.lax.dot(ohk, zb, precision=P)
        ohzj = (zj == lane128).astype(jnp.float32)
        ohzk = (zk == lane128).astype(jnp.float32)
        prej_s[...] = jax.lax.dot(ohzj, prezj_s[...], precision=P)
        prek_s[...] = jax.lax.dot(ohzk, prezk_s[...], precision=P)
        agg_s[...] = jnp.zeros((NE, SD), jnp.float32)

    x12 = prej_s[pl.ds(pblk * PBLK, PBLK), :] + prek_s[pl.ds(pblk * PBLK, PBLK), :]
    a3 = jnp.reshape(x12, (PBLK, 1, GH)) + jnp.reshape(pree_s[...], (1, NE, GH))
    a = _silu(jnp.reshape(a3, (PBLK * NE, GH)))
    h1 = _silu(jax.lax.dot(a.astype(jnp.bfloat16), peW2_ref[...],
                           preferred_element_type=jnp.float32) + peb2_ref[...])
    sc = jax.lax.dot(h1.astype(jnp.bfloat16), peW3_ref[...],
                     preferred_element_type=jnp.float32) + peb3_ref[...]
    sc3 = jnp.reshape(sc, (PBLK, NE, SD))
    wgb = jnp.reshape(wg_s[pl.ds(pblk * PBLK, PBLK), :], (PBLK, 1, SD))
    agg_s[...] += jnp.sum(sc3 * wgb, axis=0)

    @pl.when(pblk == NB - 1)
    def _final():
        t = _silu(jax.lax.dot(agg_s[...], oW1_ref[...], precision=P) + ob1_ref[...])
        out_ref[0] = jax.lax.dot(t, oW2_ref[...], precision=P) + ob2_ref[...]


@jax.jit
def kernel(h, z, pos, mask, e_feat, z_emb, peW1, peb1, peW2, peb2, peW3, peb3,
           gW1, gb1, gW2, gb2, gW3, gb3, oW1, ob1, oW2, ob2):
    posT = jnp.transpose(pos, (0, 2, 1))
    mask_f = mask.astype(jnp.float32).reshape(B, 1, N)
    maskT_f = mask.astype(jnp.float32).reshape(B, N, 1)

    codes = pl.pallas_call(
        _sel_kernel,
        grid=(B,),
        in_specs=[
            pl.BlockSpec((1, N, 3), lambda b: (b, 0, 0)),
            pl.BlockSpec((1, 3, N), lambda b: (b, 0, 0)),
            pl.BlockSpec((1, 1, N), lambda b: (b, 0, 0)),
            pl.BlockSpec((1, N, 1), lambda b: (b, 0, 0)),
        ],
        out_specs=pl.BlockSpec((1, PMAX, 1), lambda b: (b, 0, 0)),
        out_shape=jax.ShapeDtypeStruct((B, PMAX, 1), jnp.float32),
        scratch_shapes=[
            pltpu.VMEM((1, N * N), jnp.float32),
            pltpu.VMEM((1, N * N), jnp.float32),
        ],
        compiler_params=pltpu.CompilerParams(
            dimension_semantics=("arbitrary",)),
    )(pos, posT, mask_f, maskT_f)

    posp = jnp.pad(pos, ((0, 0), (0, 0), (0, 5)))
    z_col = z.astype(jnp.float32).reshape(B, N, 1)
    zemb_p = jnp.pad(z_emb, ((0, N - (MAXZ + 1)), (0, 0)))
    gW1p = jnp.pad(gW1, ((0, 360 - gW1.shape[0]), (0, 0)))
    pe_in = 3 * ZDIM
    r2 = lambda v: v.reshape(1, -1)

    wspec = lambda shape: pl.BlockSpec(shape, lambda b, p: (0,) * len(shape))
    out = pl.pallas_call(
        _agg_kernel,
        grid=(B, NB),
        in_specs=[
            pl.BlockSpec((1, PMAX, 1), lambda b, p: (b, 0, 0)),
            pl.BlockSpec((1, N, H), lambda b, p: (b, 0, 0)),
            pl.BlockSpec((1, N, 8), lambda b, p: (b, 0, 0)),
            pl.BlockSpec((1, N, 1), lambda b, p: (b, 0, 0)),
            wspec((N, ZDIM)),
            wspec((NE, EDIM)),
            wspec((pe_in, GH)), wspec((1, GH)),
            wspec((GH, GH)), wspec((1, GH)),
            wspec((GH, SD)), wspec((1, SD)),
            wspec((360, GH)), wspec((1, GH)),
            wspec((GH, GH)), wspec((1, GH)),
            wspec((GH, SD)), wspec((1, SD)),
            wspec((SD, GH)), wspec((1, GH)),
            wspec((GH, OD)), wspec((1, OD)),
        ],
        out_specs=pl.BlockSpec((1, NE, OD), lambda b, p: (b, 0, 0)),
        out_shape=jax.ShapeDtypeStruct((B, NE, OD), jnp.float32),
        scratch_shapes=[
            pltpu.VMEM((N, GH), jnp.float32),
            pltpu.VMEM((N, GH), jnp.float32),
            pltpu.VMEM((NE, GH), jnp.float32),
            pltpu.VMEM((PMAX, GH), jnp.float32),
            pltpu.VMEM((PMAX, GH), jnp.float32),
            pltpu.VMEM((PMAX, SD), jnp.float32),
            pltpu.VMEM((NE, SD), jnp.float32),
        ],
        compiler_params=pltpu.CompilerParams(
            dimension_semantics=("arbitrary", "arbitrary")),
    )(codes, h.astype(jnp.bfloat16), posp, z_col, zemb_p, e_feat,
      peW1, r2(peb1), peW2.astype(jnp.bfloat16), r2(peb2),
      peW3.astype(jnp.bfloat16), r2(peb3),
      gW1p.astype(jnp.bfloat16), r2(gb1), gW2.astype(jnp.bfloat16), r2(gb2),
      gW3.astype(jnp.bfloat16), r2(gb3),
      oW1, r2(ob1), oW2, r2(ob2))
    return out


# speedup vs baseline: 4.3232x; 1.2157x over previous
import jax
import jax.numpy as jnp
from jax.experimental import pallas as pl
from jax.experimental.pallas import tpu as pltpu

B, N, H = 8, 128, 128
NE, EDIM = 64, 32
ZDIM, MAXZ = 32, 100
RBF_DIM = 32
GH, SD, OD = 256, 128, 128
CUTOFF = 6.0
PMAX = 256
PBLK = 32
NB = PMAX // PBLK

_INF_BITS = 0x7F800000


def _silu(x):
    return x * jax.lax.logistic(x)


def _sel_kernel(pos_ref, posT_ref, mask_ref, maskT_ref, code_ref, frank_ref, fcode_ref):
    px = pos_ref[:, :, 0:1]
    py = pos_ref[:, :, 1:2]
    pz = pos_ref[:, :, 2:3]
    pxr = posT_ref[:, 0:1, :]
    pyr = posT_ref[:, 1:2, :]
    pzr = posT_ref[:, 2:3, :]
    dx = px - pxr
    dy = py - pyr
    dz = pz - pzr
    d2 = (dx * dx + dy * dy) + dz * dz
    d = jnp.sqrt(jnp.maximum(d2, 0.0))
    r0_row = d[:, 0:1, :]
    r0_col = d[:, :, 0:1]
    kIi = jax.lax.broadcasted_iota(jnp.int32, (B, N, N), 2)
    jIi = jax.lax.broadcasted_iota(jnp.int32, (B, N, N), 1)
    m_row = mask_ref[:, 0:1, :]
    m_col = maskT_ref[:, :, 0:1]
    valid_row = (m_row > 0.0) & (r0_row <= CUTOFF) & (kIi[:, 0:1, :] != 0)
    valid_col = (m_col > 0.0) & (r0_col <= CUTOFF) & (jIi[:, :, 0:1] != 0)
    pv = valid_col & valid_row & (jIi < kIi)
    score = (r0_col + r0_row) + 0.5 * d
    scoref = jnp.where(pv, score, jnp.inf)
    bits = jax.lax.bitcast_convert_type(scoref, jnp.int32)

    def bs_body(_, lohi):
        lo, hi = lohi
        mid = lo + (hi - lo) // 2
        cnt = jnp.sum((bits <= mid).astype(jnp.float32), axis=(1, 2), keepdims=True)
        ge = cnt >= float(PMAX)
        return (jnp.where(ge, lo, mid + 1), jnp.where(ge, mid, hi))

    lo0 = jnp.zeros((B, 1, 1), jnp.int32)
    hi0 = jnp.full((B, 1, 1), _INF_BITS, jnp.int32)
    _, tstar = jax.lax.fori_loop(0, 31, bs_body, (lo0, hi0))

    strict = (bits < tstar).astype(jnp.float32)
    tie = (bits == tstar).astype(jnp.float32)
    n_strict = jnp.sum(strict, axis=(1, 2), keepdims=True)

    kI2 = jax.lax.broadcasted_iota(jnp.int32, (N, N), 1)
    jI2 = jax.lax.broadcasted_iota(jnp.int32, (N, N), 0)
    up_strict = (jI2 < kI2).astype(jnp.float32)
    lo_strict = (kI2 < jI2).astype(jnp.float32)
    PX = jax.lax.Precision.HIGHEST

    def excl_rank(m):
        within = jax.lax.dot(m, up_strict, precision=PX)
        above = jax.lax.dot(lo_strict, m, precision=PX)
        row_off = jnp.sum(above, axis=1, keepdims=True)
        return row_off + within

    code2 = (jI2.astype(jnp.float32) * 512.0 + kI2.astype(jnp.float32) * 2.0)
    sl_col = jax.lax.broadcasted_iota(jnp.int32, (32, 1), 0).astype(jnp.float32)
    CH = 2048

    for b in range(B):
        tie_rank = excl_rank(tie[b])
        sel = strict[b] + tie[b] * (
            tie_rank < (float(PMAX) - n_strict[b])).astype(jnp.float32)
        rank = excl_rank(sel)
        codem = sel * (code2 + pv[b].astype(jnp.float32))
        rankm = jnp.where(sel > 0.0, rank, 1e9)
        for j in range(N):
            frank_ref[b:b + 1, j * N:(j + 1) * N] = rankm[j:j + 1, :]
            fcode_ref[b:b + 1, j * N:(j + 1) * N] = codem[j:j + 1, :]

    for b in range(B):
        acc = [jnp.zeros((32, 1), jnp.float32) for _ in range(8)]
        for blk in range(N * N // CH):
            rch = frank_ref[b:b + 1, blk * CH:(blk + 1) * CH]
            cch = fcode_ref[b:b + 1, blk * CH:(blk + 1) * CH]
            hi = jnp.floor(rch * (1.0 / 32.0))
            lo = rch - 32.0 * hi
            oh_lo = (lo == sl_col).astype(jnp.float32)
            for sh in range(8):
                msk = jnp.where(hi == float(sh), cch, 0.0)
                acc[sh] = acc[sh] + jax.lax.dot_general(
                    oh_lo, msk, (((1,), (1,)), ((), ())), precision=PX)
        for sh in range(8):
            code_ref[b, sh * 32:(sh + 1) * 32, :] = acc[sh]


def _agg_kernel(code_ref, h_ref, pos_ref, z_ref, zemb_ref, efeat_ref,
                peW1_ref, peb1_ref, peW2_ref, peb2_ref, peW3_ref, peb3_ref,
                gW1_ref, gb1_ref, gW2_ref, gb2_ref, gW3_ref, gb3_ref,
                oW1_ref, ob1_ref, oW2_ref, ob2_ref,
                out_ref,
                prezj_s, prezk_s, pree_s, prej_s, prek_s, wg_s, agg_s):
    pblk = pl.program_id(1)
    bi = pl.program_id(0)
    P = jax.lax.Precision.HIGHEST

    @pl.when((pblk == 0) & (bi == 0))
    def _tables():
        zemb = zemb_ref[...]
        prezj_s[...] = jax.lax.dot(zemb, peW1_ref[0:ZDIM, :], precision=P)
        prezk_s[...] = jax.lax.dot(zemb, peW1_ref[ZDIM:2 * ZDIM, :], precision=P)
        pree_s[...] = jax.lax.dot(efeat_ref[...], peW1_ref[2 * ZDIM:3 * ZDIM, :],
                                  precision=P) + peb1_ref[...]

    @pl.when(pblk == 0)
    def _per_batch():
        codes = code_ref[0]
        jf = jnp.floor(codes / 512.0)
        rem = codes - jf * 512.0
        kf = jnp.floor(rem / 2.0)
        vf = rem - kf * 2.0
        lane128 = jax.lax.broadcasted_iota(jnp.int32, (PMAX, N), 1).astype(jnp.float32)
        ohj = (jf == lane128).astype(jnp.float32)
        ohk = (kf == lane128).astype(jnp.float32)
        posb = pos_ref[0]
        p0 = posb[0:1, :]
        pj = jax.lax.dot(ohj, posb, precision=P)
        pk = jax.lax.dot(ohk, posb, precision=P)
        vj = pj - p0
        vk = pk - p0
        vjk = pk - pj
        r0j = jnp.sqrt(jnp.sum(vj * vj, axis=1, keepdims=True))
        r0k = jnp.sqrt(jnp.sum(vk * vk, axis=1, keepdims=True))
        rjk = jnp.sqrt(jnp.sum(vjk * vjk, axis=1, keepdims=True))
        dotjk = jnp.sum(vj * vk, axis=1, keepdims=True)
        cosang = jnp.clip(dotjk / (jnp.maximum(r0j, 1e-8) * jnp.maximum(r0k, 1e-8)),
                          -1.0, 1.0)

        delta = CUTOFF / (RBF_DIM - 1)
        gamma = 1.0 / (delta * delta + 1e-12)
        cent = jax.lax.broadcasted_iota(jnp.int32, (1, RBF_DIM), 1).astype(jnp.float32) * delta

        def rbf(r):
            rc = jnp.minimum(r, CUTOFF)
            t = rc - cent
            return jnp.exp(-gamma * (t * t))

        f0j = rbf(r0j)
        f0k = rbf(r0k)
        fjk = rbf(rjk)
        lane8 = jax.lax.broadcasted_iota(jnp.int32, (PMAX, 8), 1)
        cos8 = jnp.where(lane8 < 1, cosang, 0.0)

        bf = jnp.bfloat16
        f32 = jnp.float32
        hb = h_ref[0]
        ohj_bf = ohj.astype(bf)
        ohk_bf = ohk.astype(bf)
        hj = jax.lax.dot(ohj_bf, hb, preferred_element_type=f32)
        hk = jax.lax.dot(ohk_bf, hb, preferred_element_type=f32)
        x = (jax.lax.dot(hj.astype(bf), gW1_ref[0:128, :], preferred_element_type=f32)
             + jax.lax.dot(hk.astype(bf), gW1_ref[128:256, :], preferred_element_type=f32)
             + jax.lax.dot(f0j.astype(bf), gW1_ref[256:288, :], preferred_element_type=f32)
             + jax.lax.dot(f0k.astype(bf), gW1_ref[288:320, :], preferred_element_type=f32)
             + jax.lax.dot(fjk.astype(bf), gW1_ref[320:352, :], preferred_element_type=f32)
             + jax.lax.dot(cos8.astype(bf), gW1_ref[352:360, :], preferred_element_type=f32)
             + gb1_ref[...])
        x = _silu(x)
        x = _silu(jax.lax.dot(x.astype(bf), gW2_ref[...], preferred_element_type=f32) + gb2_ref[...])
        g = jax.lax.dot(x.astype(bf), gW3_ref[...], preferred_element_type=f32) + gb3_ref[...]

        def cut(r):
            c = 0.5 * (jnp.cos(jnp.pi * r / CUTOFF) + 1.0)
            return c * (r <= CUTOFF).astype(jnp.float32)

        env = cut(r0j) * cut(r0k) * cut(rjk) * vf
        wg_s[...] = g * env

        zb = z_ref[0]
        zj = jax.lax.dot(ohj, zb, precision=P)
        zk = jax.lax.dot(ohk, zb, precision=P)
        ohzj = (zj == lane128).astype(jnp.float32)
        ohzk = (zk == lane128).astype(jnp.float32)
        prej_s[...] = jax.lax.dot(ohzj, prezj_s[...], precision=P)
        prek_s[...] = jax.lax.dot(ohzk, prezk_s[...], precision=P)
        agg_s[...] = jnp.zeros((NE, SD), jnp.float32)

    x12 = prej_s[pl.ds(pblk * PBLK, PBLK), :] + prek_s[pl.ds(pblk * PBLK, PBLK), :]
    a3 = jnp.reshape(x12, (PBLK, 1, GH)) + jnp.reshape(pree_s[...], (1, NE, GH))
    a = _silu(jnp.reshape(a3, (PBLK * NE, GH)))
    h1 = _silu(jax.lax.dot(a.astype(jnp.bfloat16), peW2_ref[...],
                           preferred_element_type=jnp.float32) + peb2_ref[...])
    sc = jax.lax.dot(h1.astype(jnp.bfloat16), peW3_ref[...],
                     preferred_element_type=jnp.float32) + peb3_ref[...]
    sc3 = jnp.reshape(sc, (PBLK, NE, SD))
    wgb = jnp.reshape(wg_s[pl.ds(pblk * PBLK, PBLK), :], (PBLK, 1, SD))
    agg_s[...] += jnp.sum(sc3 * wgb, axis=0)

    @pl.when(pblk == NB - 1)
    def _final():
        t = _silu(jax.lax.dot(agg_s[...], oW1_ref[...], precision=P) + ob1_ref[...])
        out_ref[0] = jax.lax.dot(t, oW2_ref[...], precision=P) + ob2_ref[...]


@jax.jit
def kernel(h, z, pos, mask, e_feat, z_emb, peW1, peb1, peW2, peb2, peW3, peb3,
           gW1, gb1, gW2, gb2, gW3, gb3, oW1, ob1, oW2, ob2):
    posT = jnp.transpose(pos, (0, 2, 1))
    mask_f = mask.astype(jnp.float32).reshape(B, 1, N)
    maskT_f = mask.astype(jnp.float32).reshape(B, N, 1)

    codes = pl.pallas_call(
        _sel_kernel,
        out_shape=jax.ShapeDtypeStruct((B, PMAX, 1), jnp.float32),
        scratch_shapes=[
            pltpu.VMEM((B, N * N), jnp.float32),
            pltpu.VMEM((B, N * N), jnp.float32),
        ],
    )(pos, posT, mask_f, maskT_f)

    posp = jnp.pad(pos, ((0, 0), (0, 0), (0, 5)))
    z_col = z.astype(jnp.float32).reshape(B, N, 1)
    zemb_p = jnp.pad(z_emb, ((0, N - (MAXZ + 1)), (0, 0)))
    gW1p = jnp.pad(gW1, ((0, 360 - gW1.shape[0]), (0, 0)))
    pe_in = 3 * ZDIM
    r2 = lambda v: v.reshape(1, -1)

    wspec = lambda shape: pl.BlockSpec(shape, lambda b, p: (0,) * len(shape))
    out = pl.pallas_call(
        _agg_kernel,
        grid=(B, NB),
        in_specs=[
            pl.BlockSpec((1, PMAX, 1), lambda b, p: (b, 0, 0)),
            pl.BlockSpec((1, N, H), lambda b, p: (b, 0, 0)),
            pl.BlockSpec((1, N, 8), lambda b, p: (b, 0, 0)),
            pl.BlockSpec((1, N, 1), lambda b, p: (b, 0, 0)),
            wspec((N, ZDIM)),
            wspec((NE, EDIM)),
            wspec((pe_in, GH)), wspec((1, GH)),
            wspec((GH, GH)), wspec((1, GH)),
            wspec((GH, SD)), wspec((1, SD)),
            wspec((360, GH)), wspec((1, GH)),
            wspec((GH, GH)), wspec((1, GH)),
            wspec((GH, SD)), wspec((1, SD)),
            wspec((SD, GH)), wspec((1, GH)),
            wspec((GH, OD)), wspec((1, OD)),
        ],
        out_specs=pl.BlockSpec((1, NE, OD), lambda b, p: (b, 0, 0)),
        out_shape=jax.ShapeDtypeStruct((B, NE, OD), jnp.float32),
        scratch_shapes=[
            pltpu.VMEM((N, GH), jnp.float32),
            pltpu.VMEM((N, GH), jnp.float32),
            pltpu.VMEM((NE, GH), jnp.float32),
            pltpu.VMEM((PMAX, GH), jnp.float32),
            pltpu.VMEM((PMAX, GH), jnp.float32),
            pltpu.VMEM((PMAX, SD), jnp.float32),
            pltpu.VMEM((NE, SD), jnp.float32),
        ],
        compiler_params=pltpu.CompilerParams(
            dimension_semantics=("arbitrary", "arbitrary")),
    )(codes, h.astype(jnp.bfloat16), posp, z_col, zemb_p, e_feat,
      peW1, r2(peb1), peW2.astype(jnp.bfloat16), r2(peb2),
      peW3.astype(jnp.bfloat16), r2(peb3),
      gW1p.astype(jnp.bfloat16), r2(gb1), gW2.astype(jnp.bfloat16), r2(gb2),
      gW3.astype(jnp.bfloat16), r2(gb3),
      oW1, r2(ob1), oW2, r2(ob2))
    return out
